# Initial kernel scaffold; baseline (speedup 1.0000x reference)
import jax
import jax.numpy as jnp
from jax.experimental import pallas as pl


def kernel(x, edge_index, sW0, sb0, sW1, sb1, sW2, sb2, sW3, sb3, sW4, sb4, aW1, ab1, aW2, ab2, aW3, ab3, gWih, gWhh, gbih, gbhh, oW, ob, dW1, db1, dW2, db2):
    raise NotImplementedError("write your pallas kernel here")


# speedup vs baseline: 3.9798x; 3.9798x over previous
import functools

import jax
import jax.numpy as jnp
from jax import lax
from jax.experimental import pallas as pl
from jax.experimental.pallas import tpu as pltpu
from jax.experimental.pallas import tpu_sc as plsc

B = 1
N = 10000
T = 8
FLEN = 4
H = 128
E = 160000
TN = T * N

NC = 2
NS = 16
CHUNK = 128
NCHUNK = 40
EP = NC * NS * NCHUNK * CHUNK
STRIPE = 632
N_ACC = NS * STRIPE

_mesh = plsc.VectorSubcoreMesh(core_axis_name="c", subcore_axis_name="s")


@functools.lru_cache(maxsize=None)
def _make_scatter(D):
    @functools.partial(
        pl.kernel,
        mesh=_mesh,
        out_type=jax.ShapeDtypeStruct((NC, T, N_ACC, D), jnp.float32),
        scratch_types=[
            pltpu.VMEM((CHUNK,), jnp.int32),
            pltpu.VMEM((CHUNK,), jnp.int32),
            pltpu.VMEM((CHUNK,), jnp.int32),
            pltpu.VMEM((CHUNK, D), jnp.float32),
            pltpu.VMEM_SHARED((N_ACC, D), jnp.float32),
            pltpu.SemaphoreType.DMA,
        ],
    )
    def k(g_hbm, src_hbm, dst_hbm, zeros_hbm, out_hbm,
          src_v, idx_v, dst_v, rows_v, acc, sem):
        c = lax.axis_index("c")
        s = lax.axis_index("s")
        ebase = (c * NS + s) * (NCHUNK * CHUNK)
        rbase = s * STRIPE
        for t in range(T):
            pltpu.sync_copy(zeros_hbm, acc.at[pl.ds(rbase, STRIPE)])
            plsc.subcore_barrier()

            def body(i, carry):
                off = pl.multiple_of(ebase + i * CHUNK, CHUNK)
                pltpu.sync_copy(src_hbm.at[pl.ds(off, CHUNK)], src_v)
                pltpu.sync_copy(dst_hbm.at[pl.ds(off, CHUNK)], dst_v)
                for j in range(CHUNK // 16):
                    idx_v[pl.ds(j * 16, 16)] = src_v[pl.ds(j * 16, 16)] + t * N
                pltpu.async_copy(g_hbm.at[idx_v], rows_v, sem).wait()
                pltpu.sync_copy(rows_v, acc.at[dst_v], add=True)
                return carry

            lax.fori_loop(0, NCHUNK, body, 0)
            plsc.subcore_barrier()
            pltpu.sync_copy(acc.at[pl.ds(rbase, STRIPE)],
                            out_hbm.at[c, t, pl.ds(rbase, STRIPE)])

    return k


@functools.lru_cache(maxsize=None)
def _make_deg():
    @functools.partial(
        pl.kernel,
        mesh=_mesh,
        out_type=jax.ShapeDtypeStruct((NC, N_ACC, 128), jnp.float32),
        scratch_types=[
            pltpu.VMEM((CHUNK,), jnp.int32),
            pltpu.VMEM((CHUNK, 128), jnp.float32),
            pltpu.VMEM_SHARED((N_ACC, 128), jnp.float32),
        ],
    )
    def k(dst_hbm, ones_hbm, zeros_hbm, out_hbm, dst_v, ones_v, acc):
        c = lax.axis_index("c")
        s = lax.axis_index("s")
        ebase = (c * NS + s) * (NCHUNK * CHUNK)
        rbase = s * STRIPE
        pltpu.sync_copy(ones_hbm, ones_v)
        pltpu.sync_copy(zeros_hbm, acc.at[pl.ds(rbase, STRIPE)])
        plsc.subcore_barrier()

        def body(i, carry):
            off = pl.multiple_of(ebase + i * CHUNK, CHUNK)
            pltpu.sync_copy(dst_hbm.at[pl.ds(off, CHUNK)], dst_v)
            pltpu.sync_copy(ones_v, acc.at[dst_v], add=True)
            return carry

        lax.fori_loop(0, NCHUNK, body, 0)
        plsc.subcore_barrier()
        pltpu.sync_copy(acc.at[pl.ds(rbase, STRIPE)],
                        out_hbm.at[c, pl.ds(rbase, STRIPE)])

    return k


def _matmul(a, w, b):
    M, K = a.shape
    Dout = w.shape[1]
    bm = 800 if M % 800 == 0 else 400
    assert M % bm == 0

    def kern(a_ref, w_ref, b_ref, o_ref):
        if K == 1:
            o_ref[...] = a_ref[...] * w_ref[...] + b_ref[...]
        else:
            o_ref[...] = jnp.dot(a_ref[...], w_ref[...],
                                 preferred_element_type=jnp.float32) + b_ref[...]

    return pl.pallas_call(
        kern,
        grid=(M // bm,),
        in_specs=[
            pl.BlockSpec((bm, K), lambda i: (i, 0)),
            pl.BlockSpec((K, Dout), lambda i: (0, 0)),
            pl.BlockSpec((1, Dout), lambda i: (0, 0)),
        ],
        out_specs=pl.BlockSpec((bm, Dout), lambda i: (i, 0)),
        out_shape=jax.ShapeDtypeStruct((M, Dout), jnp.float32),
    )(a, w, b.reshape(1, Dout))


def kernel(x, edge_index, sW0, sb0, sW1, sb1, sW2, sb2, sW3, sb3, sW4, sb4,
           aW1, ab1, aW2, ab2, aW3, ab3, gWih, gWhh, gbih, gbhh, oW, ob,
           dW1, db1, dW2, db2):
    src, dst = edge_index[0], edge_index[1]
    pad = EP - E
    src_p = jnp.concatenate([src, jnp.zeros((pad,), jnp.int32)])
    dst_p = jnp.concatenate([dst, jnp.full((pad,), N, jnp.int32)])

    ones16 = jnp.ones((CHUNK, 128), jnp.float32)
    zeros16 = jnp.zeros((STRIPE, 128), jnp.float32)
    deg2 = _make_deg()(dst_p, ones16, zeros16)
    deg_sp = deg2[0, :N, 0] + deg2[1, :N, 0]
    tt = jnp.arange(T, dtype=jnp.float32)
    degs = deg_sp[None, :] + 1.0 + (tt[:, None] > 0) + (tt[:, None] < T - 1)
    u3 = (1.0 / jnp.sqrt(degs))[:, :, None]

    zcache = {}

    def gcn(h, W, b, act):
        D = W.shape[1]
        hw = _matmul(h, W, jnp.zeros((D,), jnp.float32))
        g = u3 * hw.reshape(T, N, D)
        Dp = max(D, 128)
        gs = g.reshape(TN, D)
        if Dp != D:
            gs = jnp.pad(gs, ((0, 0), (0, Dp - D)))
        if Dp not in zcache:
            zcache[Dp] = jnp.zeros((STRIPE, Dp), jnp.float32)
        S2 = _make_scatter(Dp)(gs, src_p, dst_p, zcache[Dp])
        S = S2[0, :, :N, :D] + S2[1, :, :N, :D]
        z1 = jnp.zeros((1, N, D), jnp.float32)
        gm = jnp.concatenate([z1, g[:-1]], 0)
        gp = jnp.concatenate([g[1:], z1], 0)
        out = u3 * (S + gm + gp + g) + b
        return act(out).reshape(TN, D)

    h = jnp.transpose(x, (0, 2, 1)).reshape(-1, 1)
    for W, b in ((sW0, sb0), (sW1, sb1), (sW2, sb2), (sW3, sb3), (sW4, sb4)):
        h = gcn(h, W, b, jax.nn.relu)

    a1 = jnp.tanh(_matmul(h, aW1, ab1))
    a2 = jnp.tanh(_matmul(a1, aW2, ab2))
    a3 = _matmul(a2, aW3, ab3).reshape(T, N)
    aw = jax.nn.softmax(a3, axis=0)
    nf = jnp.sum(h.reshape(T, N, H) * aw[:, :, None], axis=0)

    gi = _matmul(nf, gWih.T, gbih)
    ir, iz, inn = jnp.split(gi, 3, axis=-1)
    hr, hz, hn = jnp.split(gbhh, 3)
    z = jax.nn.sigmoid(iz + hz)
    n = jnp.tanh(inn + jax.nn.sigmoid(ir + hr) * hn)
    hid = (1.0 - z) * n
    y0 = jnp.repeat(hid, T, axis=0)

    def odef(y):
        y1 = gcn(y, dW1, db1, jnp.tanh)
        return gcn(y1, dW2, db2, jnp.tanh)

    ts = jnp.linspace(0.0, float(FLEN), FLEN)
    ys = [y0]
    y = y0
    for i in range(FLEN - 1):
        dt = ts[i + 1] - ts[i]
        k1 = odef(y)
        k2 = odef(y + dt * k1 / 3.0)
        k3 = odef(y + dt * (k2 - k1 / 3.0))
        k4 = odef(y + dt * (k1 - k2 + k3))
        y = y + dt * (k1 + 3.0 * (k2 + k3) + k4) * 0.125
        ys.append(y)

    evolved = jnp.stack([yy[(T - 1) * N:] for yy in ys], 0)
    pred = _matmul(evolved.reshape(FLEN * N, H), oW, ob)
    return jnp.transpose(pred.reshape(FLEN, N), (1, 0))[None]


# speedup vs baseline: 4.8273x; 1.2130x over previous
import functools

import jax
import jax.numpy as jnp
from jax import lax
from jax.experimental import pallas as pl
from jax.experimental.pallas import tpu as pltpu
from jax.experimental.pallas import tpu_sc as plsc

B = 1
N = 10000
T = 8
FLEN = 4
H = 128
E = 160000
TN = T * N

NC = 2
NS = 16
CHUNK = 128
NCHUNK = 40
EP = NC * NS * NCHUNK * CHUNK
STRIPE = 632
N_ACC = NS * STRIPE

_mesh = plsc.VectorSubcoreMesh(core_axis_name="c", subcore_axis_name="s")


@functools.lru_cache(maxsize=None)
def _make_scatter(D):
    @functools.partial(
        pl.kernel,
        mesh=_mesh,
        out_type=jax.ShapeDtypeStruct((NC, T, N_ACC, D), jnp.float32),
        scratch_types=[
            pltpu.VMEM((NCHUNK, CHUNK), jnp.int32),
            pltpu.VMEM((NCHUNK, CHUNK), jnp.int32),
            pltpu.VMEM((NCHUNK, CHUNK), jnp.int32),
            pltpu.VMEM((CHUNK, D), jnp.float32),
            pltpu.VMEM((CHUNK, D), jnp.float32),
            pltpu.VMEM_SHARED((N_ACC, D), jnp.float32),
            pltpu.SemaphoreType.DMA,
            pltpu.SemaphoreType.DMA,
        ],
    )
    def k(g_hbm, src_hbm, dst_hbm, zeros_hbm, out_hbm,
          src_v, idx_v, dst_v, rows0, rows1, acc, sem0, sem1):
        c = lax.axis_index("c")
        s = lax.axis_index("s")
        wid = c * NS + s
        rbase = s * STRIPE
        pltpu.sync_copy(src_hbm.at[pl.ds(wid * NCHUNK, NCHUNK)], src_v)
        pltpu.sync_copy(dst_hbm.at[pl.ds(wid * NCHUNK, NCHUNK)], dst_v)
        for t in range(T):
            def ibody(q, carry):
                for j in range(CHUNK // 16):
                    idx_v[q, pl.ds(j * 16, 16)] = (
                        src_v[q, pl.ds(j * 16, 16)] + t * N)
                return carry

            lax.fori_loop(0, NCHUNK, ibody, 0)
            pltpu.sync_copy(zeros_hbm, acc.at[pl.ds(rbase, STRIPE)])
            plsc.subcore_barrier()

            pltpu.async_copy(g_hbm.at[idx_v.at[0]], rows0, sem0)

            def body(p, carry):
                i1 = p * 2 + 1
                pltpu.async_copy(g_hbm.at[idx_v.at[i1]], rows1, sem1)
                pltpu.make_async_copy(g_hbm.at[idx_v.at[0]], rows0, sem0).wait()
                pltpu.sync_copy(rows0, acc.at[dst_v.at[p * 2]], add=True)

                @pl.when(i1 + 1 < NCHUNK)
                def _():
                    pltpu.async_copy(g_hbm.at[idx_v.at[i1 + 1]], rows0, sem0)

                pltpu.make_async_copy(g_hbm.at[idx_v.at[0]], rows1, sem1).wait()
                pltpu.sync_copy(rows1, acc.at[dst_v.at[i1]], add=True)
                return carry

            lax.fori_loop(0, NCHUNK // 2, body, 0)
            plsc.subcore_barrier()
            pltpu.sync_copy(acc.at[pl.ds(rbase, STRIPE)],
                            out_hbm.at[c, t, pl.ds(rbase, STRIPE)])

    return k


@functools.lru_cache(maxsize=None)
def _make_deg():
    @functools.partial(
        pl.kernel,
        mesh=_mesh,
        out_type=jax.ShapeDtypeStruct((NC, N_ACC, 128), jnp.float32),
        scratch_types=[
            pltpu.VMEM((CHUNK,), jnp.int32),
            pltpu.VMEM((CHUNK, 128), jnp.float32),
            pltpu.VMEM_SHARED((N_ACC, 128), jnp.float32),
        ],
    )
    def k(dst_hbm, ones_hbm, zeros_hbm, out_hbm, dst_v, ones_v, acc):
        c = lax.axis_index("c")
        s = lax.axis_index("s")
        ebase = (c * NS + s) * (NCHUNK * CHUNK)
        rbase = s * STRIPE
        pltpu.sync_copy(ones_hbm, ones_v)
        pltpu.sync_copy(zeros_hbm, acc.at[pl.ds(rbase, STRIPE)])
        plsc.subcore_barrier()

        def body(i, carry):
            off = pl.multiple_of(ebase + i * CHUNK, CHUNK)
            pltpu.sync_copy(dst_hbm.at[pl.ds(off, CHUNK)], dst_v)
            pltpu.sync_copy(ones_v, acc.at[dst_v], add=True)
            return carry

        lax.fori_loop(0, NCHUNK, body, 0)
        plsc.subcore_barrier()
        pltpu.sync_copy(acc.at[pl.ds(rbase, STRIPE)],
                        out_hbm.at[c, pl.ds(rbase, STRIPE)])

    return k


def _matmul(a, w, b):
    M, K = a.shape
    Dout = w.shape[1]
    bm = 800 if M % 800 == 0 else 400
    assert M % bm == 0

    def kern(a_ref, w_ref, b_ref, o_ref):
        if K == 1:
            o_ref[...] = a_ref[...] * w_ref[...] + b_ref[...]
        else:
            o_ref[...] = jnp.dot(a_ref[...], w_ref[...],
                                 preferred_element_type=jnp.float32) + b_ref[...]

    return pl.pallas_call(
        kern,
        grid=(M // bm,),
        in_specs=[
            pl.BlockSpec((bm, K), lambda i: (i, 0)),
            pl.BlockSpec((K, Dout), lambda i: (0, 0)),
            pl.BlockSpec((1, Dout), lambda i: (0, 0)),
        ],
        out_specs=pl.BlockSpec((bm, Dout), lambda i: (i, 0)),
        out_shape=jax.ShapeDtypeStruct((M, Dout), jnp.float32),
    )(a, w, b.reshape(1, Dout))


def kernel(x, edge_index, sW0, sb0, sW1, sb1, sW2, sb2, sW3, sb3, sW4, sb4,
           aW1, ab1, aW2, ab2, aW3, ab3, gWih, gWhh, gbih, gbhh, oW, ob,
           dW1, db1, dW2, db2):
    src, dst = edge_index[0], edge_index[1]
    pad = EP - E
    src_p = jnp.concatenate([src, jnp.zeros((pad,), jnp.int32)])
    dst_p = jnp.concatenate([dst, jnp.full((pad,), N, jnp.int32)])
    src_2d = src_p.reshape(EP // CHUNK, CHUNK)
    dst_2d = dst_p.reshape(EP // CHUNK, CHUNK)

    ones16 = jnp.ones((CHUNK, 128), jnp.float32)
    zeros16 = jnp.zeros((STRIPE, 128), jnp.float32)
    deg2 = _make_deg()(dst_p, ones16, zeros16)
    deg_sp = deg2[0, :N, 0] + deg2[1, :N, 0]
    tt = jnp.arange(T, dtype=jnp.float32)
    degs = deg_sp[None, :] + 1.0 + (tt[:, None] > 0) + (tt[:, None] < T - 1)
    u3 = (1.0 / jnp.sqrt(degs))[:, :, None]

    zcache = {}

    def gcn(h, W, b, act):
        D = W.shape[1]
        hw = _matmul(h, W, jnp.zeros((D,), jnp.float32))
        g = u3 * hw.reshape(T, N, D)
        Dp = max(D, 128)
        gs = g.reshape(TN, D)
        if Dp != D:
            gs = jnp.pad(gs, ((0, 0), (0, Dp - D)))
        if Dp not in zcache:
            zcache[Dp] = jnp.zeros((STRIPE, Dp), jnp.float32)
        S2 = _make_scatter(Dp)(gs, src_2d, dst_2d, zcache[Dp])
        S = S2[0, :, :N, :D] + S2[1, :, :N, :D]
        z1 = jnp.zeros((1, N, D), jnp.float32)
        gm = jnp.concatenate([z1, g[:-1]], 0)
        gp = jnp.concatenate([g[1:], z1], 0)
        out = u3 * (S + gm + gp + g) + b
        return act(out).reshape(TN, D)

    h = jnp.transpose(x, (0, 2, 1)).reshape(-1, 1)
    for W, b in ((sW0, sb0), (sW1, sb1), (sW2, sb2), (sW3, sb3), (sW4, sb4)):
        h = gcn(h, W, b, jax.nn.relu)

    a1 = jnp.tanh(_matmul(h, aW1, ab1))
    a2 = jnp.tanh(_matmul(a1, aW2, ab2))
    a3 = _matmul(a2, aW3, ab3).reshape(T, N)
    aw = jax.nn.softmax(a3, axis=0)
    nf = jnp.sum(h.reshape(T, N, H) * aw[:, :, None], axis=0)

    gi = _matmul(nf, gWih.T, gbih)
    ir, iz, inn = jnp.split(gi, 3, axis=-1)
    hr, hz, hn = jnp.split(gbhh, 3)
    z = jax.nn.sigmoid(iz + hz)
    n = jnp.tanh(inn + jax.nn.sigmoid(ir + hr) * hn)
    hid = (1.0 - z) * n
    y0 = jnp.repeat(hid, T, axis=0)

    def odef(y):
        y1 = gcn(y, dW1, db1, jnp.tanh)
        return gcn(y1, dW2, db2, jnp.tanh)

    ts = jnp.linspace(0.0, float(FLEN), FLEN)
    ys = [y0]
    y = y0
    for i in range(FLEN - 1):
        dt = ts[i + 1] - ts[i]
        k1 = odef(y)
        k2 = odef(y + dt * k1 / 3.0)
        k3 = odef(y + dt * (k2 - k1 / 3.0))
        k4 = odef(y + dt * (k1 - k2 + k3))
        y = y + dt * (k1 + 3.0 * (k2 + k3) + k4) * 0.125
        ys.append(y)

    evolved = jnp.stack([yy[(T - 1) * N:] for yy in ys], 0)
    pred = _matmul(evolved.reshape(FLEN * N, H), oW, ob)
    return jnp.transpose(pred.reshape(FLEN, N), (1, 0))[None]
